# Initial kernel scaffold; baseline (speedup 1.0000x reference)
import jax
import jax.numpy as jnp
from jax.experimental import pallas as pl


def kernel(x, edge_index, batch, wc0, bc0, wc1, bc1, wf0, bf0, wf1, bf1):
    raise NotImplementedError("write your pallas kernel here")


# speedup vs baseline: 1.0284x; 1.0284x over previous
import functools

import jax
import jax.numpy as jnp
from jax.experimental import pallas as pl
from jax.experimental.pallas import tpu as pltpu

_GA = 128
_NCOL = 66


def _conv_pool_kernel(x_ref, b_ref, wc0t_ref, bc0_ref, wc1t_ref, bc1_ref,
                      out_ref):
    i = pl.program_id(1)

    @pl.when(i == 0)
    def _init():
        out_ref[...] = jnp.zeros_like(out_ref)

    t = x_ref.shape[0]
    h = jnp.dot(x_ref[...], wc0t_ref[...],
                preferred_element_type=jnp.float32) + bc0_ref[...]
    h = jnp.maximum(h, 0.0)
    h2 = jnp.dot(h.astype(jnp.bfloat16), wc1t_ref[...],
                 preferred_element_type=jnp.float32) + bc1_ref[...]
    h2 = jnp.maximum(h2, 0.0)

    b = b_ref[...]
    a = jnp.bitwise_and(b, _GA - 1)
    s = jnp.right_shift(b, 7)
    lane = jax.lax.broadcasted_iota(jnp.int32, (t, _GA), 1)
    oa = (lane == a).astype(jnp.float32).astype(jnp.bfloat16)
    hw = h2.shape[1]
    sb = jnp.broadcast_to(s, (t, hw))
    zero = jnp.zeros_like(h2)
    c0 = (s == 0).astype(jnp.float32)
    c1 = (s == 1).astype(jnp.float32)
    tmp = jnp.concatenate([
        jnp.where(sb == 0, h2, zero),
        jnp.where(sb == 1, h2, zero),
        c0,
        c1,
    ], axis=1).astype(jnp.bfloat16)
    out_ref[...] += jax.lax.dot_general(
        oa, tmp, (((0,), (0,)), ((), ())),
        preferred_element_type=jnp.float32)


def _head_kernel(p_ref, wf0t_ref, bf0_ref, wf1t_ref, bf1_ref, out_ref):
    p = jnp.sum(p_ref[...], axis=0)
    pooled = jnp.concatenate([p[:, 0:32], p[:, 32:64]], axis=0)
    counts = jnp.concatenate([p[:, 64:65], p[:, 65:66]], axis=0)
    pooled = pooled * (1.0 / jnp.maximum(counts, 1.0))
    h = jnp.dot(pooled, wf0t_ref[...],
                preferred_element_type=jnp.float32) + bf0_ref[...]
    h = jnp.maximum(h, 0.0)
    out_ref[...] = jnp.dot(h, wf1t_ref[...],
                           preferred_element_type=jnp.float32) + bf1_ref[...]


@functools.partial(jax.jit, static_argnames=("node_tile", "num_cores"))
def _forward(x, batch, wc0, bc0, wc1, bc1, wf0, bf0, wf1, bf1,
             node_tile=4096, num_cores=2):
    n, cin = x.shape
    hdim = wc0.shape[0]
    hfc = wf1.shape[0]

    n_pad = ((n + num_cores * node_tile - 1)
             // (num_cores * node_tile)) * (num_cores * node_tile)
    tiles_per_core = n_pad // (num_cores * node_tile)

    x_p = jnp.pad(x, ((0, n_pad - n), (0, 0)))
    b_p = jnp.pad(batch.astype(jnp.int32), (0, n_pad - n),
                  constant_values=2 * _GA).reshape(n_pad, 1)

    wc0t = wc0.T.astype(jnp.float32)
    bc0r = bc0.astype(jnp.float32).reshape(1, hdim)
    wc1t = wc1.T.astype(jnp.bfloat16)
    bc1r = bc1.astype(jnp.float32).reshape(1, hdim)

    flops = int(2 * n_pad * (cin * hdim + hdim * hdim + _GA * _NCOL))
    bytes_accessed = int(x_p.size * 4 + b_p.size * 4
                         + num_cores * _GA * _NCOL * 4)
    cost = pl.CostEstimate(flops=flops, transcendentals=0,
                           bytes_accessed=bytes_accessed)

    partials = pl.pallas_call(
        _conv_pool_kernel,
        out_shape=jax.ShapeDtypeStruct((num_cores, _GA, _NCOL), jnp.float32),
        grid=(num_cores, tiles_per_core),
        in_specs=[
            pl.BlockSpec((node_tile, cin),
                         lambda c, i: (c * tiles_per_core + i, 0)),
            pl.BlockSpec((node_tile, 1),
                         lambda c, i: (c * tiles_per_core + i, 0)),
            pl.BlockSpec((cin, hdim), lambda c, i: (0, 0)),
            pl.BlockSpec((1, hdim), lambda c, i: (0, 0)),
            pl.BlockSpec((hdim, hdim), lambda c, i: (0, 0)),
            pl.BlockSpec((1, hdim), lambda c, i: (0, 0)),
        ],
        out_specs=pl.BlockSpec((None, _GA, _NCOL), lambda c, i: (c, 0, 0)),
        compiler_params=pltpu.CompilerParams(
            dimension_semantics=("parallel", "arbitrary"),
            vmem_limit_bytes=64 * 1024 * 1024,
        ),
        cost_estimate=cost,
    )(x_p, b_p, wc0t, bc0r, wc1t, bc1r)

    out = pl.pallas_call(
        _head_kernel,
        out_shape=jax.ShapeDtypeStruct((2 * _GA, hfc), jnp.float32),
    )(partials, wf0.T.astype(jnp.float32),
      bf0.astype(jnp.float32).reshape(1, hfc),
      wf1.T.astype(jnp.float32),
      bf1.astype(jnp.float32).reshape(1, hfc))
    return out


def kernel(x, edge_index, batch, wc0, bc0, wc1, bc1, wf0, bf0, wf1, bf1):
    del edge_index
    return _forward(x, batch, wc0, bc0, wc1, bc1, wf0, bf0, wf1, bf1)


# speedup vs baseline: 5.4579x; 5.3071x over previous
import functools

import jax
import jax.numpy as jnp
from jax.experimental import pallas as pl
from jax.experimental.pallas import tpu as pltpu

_GA = 128
_NCOL = 66


def _conv_pool_kernel(x_ref, b_ref, wc0_ref, bc0_ref, wc1_ref, bc1_ref,
                      out_ref):
    i = pl.program_id(1)

    @pl.when(i == 0)
    def _init():
        out_ref[...] = jnp.zeros_like(out_ref)

    t = x_ref.shape[1]
    h = jnp.dot(wc0_ref[...], x_ref[...],
                preferred_element_type=jnp.float32) + bc0_ref[...]
    h = jnp.maximum(h, 0.0)
    h2 = jnp.dot(wc1_ref[...], h.astype(jnp.bfloat16),
                 preferred_element_type=jnp.float32) + bc1_ref[...]
    h2 = jnp.maximum(h2, 0.0)

    b = b_ref[...]
    a = jnp.bitwise_and(b, _GA - 1)
    s = jnp.right_shift(b, 7)
    sub = jax.lax.broadcasted_iota(jnp.int32, (_GA, t), 0)
    oa = (sub == a).astype(jnp.float32).astype(jnp.bfloat16)
    s0 = (s == 0).astype(jnp.float32)
    s1 = (s == 1).astype(jnp.float32)
    tmp = jnp.concatenate([
        h2 * s0,
        h2 * s1,
        s0,
        s1,
    ], axis=0).astype(jnp.bfloat16)
    out_ref[...] += jax.lax.dot_general(
        oa, tmp, (((1,), (1,)), ((), ())),
        preferred_element_type=jnp.float32)


def _head_kernel(p_ref, wf0t_ref, bf0_ref, wf1t_ref, bf1_ref, out_ref):
    p = jnp.sum(p_ref[...], axis=0)
    pooled = jnp.concatenate([p[:, 0:32], p[:, 32:64]], axis=0)
    counts = jnp.concatenate([p[:, 64:65], p[:, 65:66]], axis=0)
    pooled = pooled * (1.0 / jnp.maximum(counts, 1.0))
    h = jnp.dot(pooled, wf0t_ref[...],
                preferred_element_type=jnp.float32) + bf0_ref[...]
    h = jnp.maximum(h, 0.0)
    out_ref[...] = jnp.dot(h, wf1t_ref[...],
                           preferred_element_type=jnp.float32) + bf1_ref[...]


@functools.partial(jax.jit, static_argnames=("node_tile", "num_cores"))
def _forward(x, batch, wc0, bc0, wc1, bc1, wf0, bf0, wf1, bf1,
             node_tile=4096, num_cores=2):
    n, cin = x.shape
    hdim = wc0.shape[0]
    hfc = wf1.shape[0]

    n_pad = ((n + num_cores * node_tile - 1)
             // (num_cores * node_tile)) * (num_cores * node_tile)
    tiles_per_core = n_pad // (num_cores * node_tile)

    x_t = x.T
    b_r = batch.astype(jnp.int32).reshape(1, n)
    if n_pad != n:
        x_t = jnp.pad(x_t, ((0, 0), (0, n_pad - n)))
        b_r = jnp.pad(b_r, ((0, 0), (0, n_pad - n)), constant_values=2 * _GA)

    wc0f = wc0.astype(jnp.float32)
    bc0c = bc0.astype(jnp.float32).reshape(hdim, 1)
    wc1b = wc1.astype(jnp.bfloat16)
    bc1c = bc1.astype(jnp.float32).reshape(hdim, 1)

    flops = int(2 * n_pad * (cin * hdim + hdim * hdim + _GA * _NCOL))
    bytes_accessed = int(x_t.size * 4 + b_r.size * 4
                         + num_cores * _GA * _NCOL * 4)
    cost = pl.CostEstimate(flops=flops, transcendentals=0,
                           bytes_accessed=bytes_accessed)

    partials = pl.pallas_call(
        _conv_pool_kernel,
        out_shape=jax.ShapeDtypeStruct((num_cores, _GA, _NCOL), jnp.float32),
        grid=(num_cores, tiles_per_core),
        in_specs=[
            pl.BlockSpec((cin, node_tile),
                         lambda c, i: (0, c * tiles_per_core + i)),
            pl.BlockSpec((1, node_tile),
                         lambda c, i: (0, c * tiles_per_core + i)),
            pl.BlockSpec((hdim, cin), lambda c, i: (0, 0)),
            pl.BlockSpec((hdim, 1), lambda c, i: (0, 0)),
            pl.BlockSpec((hdim, hdim), lambda c, i: (0, 0)),
            pl.BlockSpec((hdim, 1), lambda c, i: (0, 0)),
        ],
        out_specs=pl.BlockSpec((None, _GA, _NCOL), lambda c, i: (c, 0, 0)),
        compiler_params=pltpu.CompilerParams(
            dimension_semantics=("parallel", "arbitrary"),
            vmem_limit_bytes=64 * 1024 * 1024,
        ),
        cost_estimate=cost,
    )(x_t, b_r, wc0f, bc0c, wc1b, bc1c)

    out = pl.pallas_call(
        _head_kernel,
        out_shape=jax.ShapeDtypeStruct((2 * _GA, hfc), jnp.float32),
    )(partials, wf0.T.astype(jnp.float32),
      bf0.astype(jnp.float32).reshape(1, hfc),
      wf1.T.astype(jnp.float32),
      bf1.astype(jnp.float32).reshape(1, hfc))
    return out


def kernel(x, edge_index, batch, wc0, bc0, wc1, bc1, wf0, bf0, wf1, bf1):
    del edge_index
    return _forward(x, batch, wc0, bc0, wc1, bc1, wf0, bf0, wf1, bf1)


# speedup vs baseline: 7.8594x; 1.4400x over previous
import functools

import jax
import jax.numpy as jnp
from jax.experimental import pallas as pl
from jax.experimental.pallas import tpu as pltpu

_GA = 128
_NCOL = 66


def _conv_pool_kernel(x_ref, b_ref, wc0_ref, bc0_ref, wc1_ref, bc1_ref,
                      out_ref):
    i = pl.program_id(0)

    @pl.when(i == 0)
    def _init():
        out_ref[...] = jnp.zeros_like(out_ref)

    t = x_ref.shape[1]
    h = jnp.dot(wc0_ref[...], x_ref[...].astype(jnp.bfloat16),
                preferred_element_type=jnp.float32) + bc0_ref[...]
    h = jnp.maximum(h, 0.0)
    h2 = jnp.dot(wc1_ref[...], h.astype(jnp.bfloat16),
                 preferred_element_type=jnp.float32) + bc1_ref[...]
    h2 = jnp.maximum(h2, 0.0)

    b = b_ref[...]
    a = jnp.bitwise_and(b, _GA - 1)
    s = jnp.right_shift(b, 7)
    sub = jax.lax.broadcasted_iota(jnp.int32, (_GA, t), 0)
    oa = (sub == a).astype(jnp.bfloat16)
    s0 = (s == 0).astype(jnp.float32)
    s1 = (s == 1).astype(jnp.float32)
    tmp = jnp.concatenate([
        h2 * s0,
        h2 * s1,
        s0,
        s1,
    ], axis=0).astype(jnp.bfloat16)
    out_ref[...] += jax.lax.dot_general(
        oa, tmp, (((1,), (1,)), ((), ())),
        preferred_element_type=jnp.float32)


def _head_kernel(p_ref, wf0t_ref, bf0_ref, wf1t_ref, bf1_ref, out_ref):
    p = p_ref[...]
    pooled = jnp.concatenate([p[:, 0:32], p[:, 32:64]], axis=0)
    counts = jnp.concatenate([p[:, 64:65], p[:, 65:66]], axis=0)
    pooled = pooled * (1.0 / jnp.maximum(counts, 1.0))
    h = jnp.dot(pooled, wf0t_ref[...],
                preferred_element_type=jnp.float32) + bf0_ref[...]
    h = jnp.maximum(h, 0.0)
    out_ref[...] = jnp.dot(h, wf1t_ref[...],
                           preferred_element_type=jnp.float32) + bf1_ref[...]


@functools.partial(jax.jit, static_argnames=("node_tile",))
def _forward(x, batch, wc0, bc0, wc1, bc1, wf0, bf0, wf1, bf1,
             node_tile=8192):
    n, cin = x.shape
    hdim = wc0.shape[0]
    hfc = wf1.shape[0]

    n_pad = ((n + node_tile - 1) // node_tile) * node_tile
    tiles = n_pad // node_tile

    x_t = x.T
    b_r = batch.astype(jnp.int32).reshape(1, n)
    if n_pad != n:
        x_t = jnp.pad(x_t, ((0, 0), (0, n_pad - n)))
        b_r = jnp.pad(b_r, ((0, 0), (0, n_pad - n)), constant_values=2 * _GA)

    wc0b = wc0.astype(jnp.bfloat16)
    bc0c = bc0.astype(jnp.float32).reshape(hdim, 1)
    wc1b = wc1.astype(jnp.bfloat16)
    bc1c = bc1.astype(jnp.float32).reshape(hdim, 1)

    flops = int(2 * n_pad * (cin * hdim + hdim * hdim + _GA * _NCOL))
    bytes_accessed = int(x_t.size * 4 + b_r.size * 4 + _GA * _NCOL * 4)
    cost = pl.CostEstimate(flops=flops, transcendentals=0,
                           bytes_accessed=bytes_accessed)

    partials = pl.pallas_call(
        _conv_pool_kernel,
        out_shape=jax.ShapeDtypeStruct((_GA, _NCOL), jnp.float32),
        grid=(tiles,),
        in_specs=[
            pl.BlockSpec((cin, node_tile), lambda i: (0, i)),
            pl.BlockSpec((1, node_tile), lambda i: (0, i)),
            pl.BlockSpec((hdim, cin), lambda i: (0, 0)),
            pl.BlockSpec((hdim, 1), lambda i: (0, 0)),
            pl.BlockSpec((hdim, hdim), lambda i: (0, 0)),
            pl.BlockSpec((hdim, 1), lambda i: (0, 0)),
        ],
        out_specs=pl.BlockSpec((_GA, _NCOL), lambda i: (0, 0)),
        compiler_params=pltpu.CompilerParams(
            dimension_semantics=("arbitrary",),
            vmem_limit_bytes=64 * 1024 * 1024,
        ),
        cost_estimate=cost,
    )(x_t, b_r, wc0b, bc0c, wc1b, bc1c)

    out = pl.pallas_call(
        _head_kernel,
        out_shape=jax.ShapeDtypeStruct((2 * _GA, hfc), jnp.float32),
    )(partials, wf0.T.astype(jnp.float32),
      bf0.astype(jnp.float32).reshape(1, hfc),
      wf1.T.astype(jnp.float32),
      bf1.astype(jnp.float32).reshape(1, hfc))
    return out


def kernel(x, edge_index, batch, wc0, bc0, wc1, bc1, wf0, bf0, wf1, bf1):
    del edge_index
    return _forward(x, batch, wc0, bc0, wc1, bc1, wf0, bf0, wf1, bf1)


# speedup vs baseline: 8.4748x; 1.0783x over previous
import functools

import jax
import jax.numpy as jnp
from jax.experimental import pallas as pl
from jax.experimental.pallas import tpu as pltpu

_GA = 128
_NCOL = 66


def _conv_pool_kernel(x_ref, b_ref, wc0_ref, bc0_ref, wc1_ref, bc1_ref,
                      out_ref):
    i = pl.program_id(0)

    @pl.when(i == 0)
    def _init():
        out_ref[...] = jnp.zeros_like(out_ref)

    t = x_ref.shape[1]
    h = jnp.dot(wc0_ref[...], x_ref[...].astype(jnp.bfloat16),
                preferred_element_type=jnp.float32) + bc0_ref[...]
    h = jnp.maximum(h, 0.0)
    h2 = jnp.dot(wc1_ref[...], h.astype(jnp.bfloat16),
                 preferred_element_type=jnp.float32) + bc1_ref[...]
    h2 = jnp.maximum(h2, 0.0)

    b = b_ref[...]
    a = jnp.bitwise_and(b, _GA - 1)
    s = jnp.right_shift(b, 7)
    sub = jax.lax.broadcasted_iota(jnp.int32, (_GA, t), 0)
    oa = (sub == a).astype(jnp.bfloat16)
    s0 = (s == 0).astype(jnp.float32)
    s1 = (s == 1).astype(jnp.float32)
    tmp = jnp.concatenate([
        h2 * s0,
        h2 * s1,
        s0,
        s1,
    ], axis=0).astype(jnp.bfloat16)
    out_ref[...] += jax.lax.dot_general(
        oa, tmp, (((1,), (1,)), ((), ())),
        preferred_element_type=jnp.float32)


def _head_kernel(p_ref, wf0t_ref, bf0_ref, wf1t_ref, bf1_ref, out_ref):
    p = p_ref[...]
    pooled = jnp.concatenate([p[:, 0:32], p[:, 32:64]], axis=0)
    counts = jnp.concatenate([p[:, 64:65], p[:, 65:66]], axis=0)
    pooled = pooled * (1.0 / jnp.maximum(counts, 1.0))
    h = jnp.dot(pooled, wf0t_ref[...],
                preferred_element_type=jnp.float32) + bf0_ref[...]
    h = jnp.maximum(h, 0.0)
    out_ref[...] = jnp.dot(h, wf1t_ref[...],
                           preferred_element_type=jnp.float32) + bf1_ref[...]


@functools.partial(jax.jit, static_argnames=("node_tile",))
def _forward(x, batch, wc0, bc0, wc1, bc1, wf0, bf0, wf1, bf1,
             node_tile=16384):
    n, cin = x.shape
    hdim = wc0.shape[0]
    hfc = wf1.shape[0]

    n_pad = ((n + node_tile - 1) // node_tile) * node_tile
    tiles = n_pad // node_tile

    x_t = x.T
    b_r = batch.astype(jnp.int32).reshape(1, n)
    if n_pad != n:
        x_t = jnp.pad(x_t, ((0, 0), (0, n_pad - n)))
        b_r = jnp.pad(b_r, ((0, 0), (0, n_pad - n)), constant_values=2 * _GA)

    wc0b = wc0.astype(jnp.bfloat16)
    bc0c = bc0.astype(jnp.float32).reshape(hdim, 1)
    wc1b = wc1.astype(jnp.bfloat16)
    bc1c = bc1.astype(jnp.float32).reshape(hdim, 1)

    flops = int(2 * n_pad * (cin * hdim + hdim * hdim + _GA * _NCOL))
    bytes_accessed = int(x_t.size * 4 + b_r.size * 4 + _GA * _NCOL * 4)
    cost = pl.CostEstimate(flops=flops, transcendentals=0,
                           bytes_accessed=bytes_accessed)

    partials = pl.pallas_call(
        _conv_pool_kernel,
        out_shape=jax.ShapeDtypeStruct((_GA, _NCOL), jnp.float32),
        grid=(tiles,),
        in_specs=[
            pl.BlockSpec((cin, node_tile), lambda i: (0, i)),
            pl.BlockSpec((1, node_tile), lambda i: (0, i)),
            pl.BlockSpec((hdim, cin), lambda i: (0, 0)),
            pl.BlockSpec((hdim, 1), lambda i: (0, 0)),
            pl.BlockSpec((hdim, hdim), lambda i: (0, 0)),
            pl.BlockSpec((hdim, 1), lambda i: (0, 0)),
        ],
        out_specs=pl.BlockSpec((_GA, _NCOL), lambda i: (0, 0)),
        compiler_params=pltpu.CompilerParams(
            dimension_semantics=("arbitrary",),
            vmem_limit_bytes=64 * 1024 * 1024,
        ),
        cost_estimate=cost,
    )(x_t, b_r, wc0b, bc0c, wc1b, bc1c)

    out = pl.pallas_call(
        _head_kernel,
        out_shape=jax.ShapeDtypeStruct((2 * _GA, hfc), jnp.float32),
    )(partials, wf0.T.astype(jnp.float32),
      bf0.astype(jnp.float32).reshape(1, hfc),
      wf1.T.astype(jnp.float32),
      bf1.astype(jnp.float32).reshape(1, hfc))
    return out


def kernel(x, edge_index, batch, wc0, bc0, wc1, bc1, wf0, bf0, wf1, bf1):
    del edge_index
    return _forward(x, batch, wc0, bc0, wc1, bc1, wf0, bf0, wf1, bf1)


# speedup vs baseline: 8.8024x; 1.0387x over previous
import functools

import jax
import jax.numpy as jnp
from jax.experimental import pallas as pl
from jax.experimental.pallas import tpu as pltpu

_GA = 128
_NCOL = 66


def _conv_pool_kernel(x_ref, b_ref, wc0_ref, bc0_ref, wc1_ref, bc1_ref,
                      out_ref):
    i = pl.program_id(0)

    @pl.when(i == 0)
    def _init():
        out_ref[...] = jnp.zeros_like(out_ref)

    t = x_ref.shape[1]
    h = jnp.dot(wc0_ref[...], x_ref[...].astype(jnp.bfloat16),
                preferred_element_type=jnp.float32) + bc0_ref[...]
    h = jnp.maximum(h, 0.0)
    h2 = jnp.dot(wc1_ref[...], h.astype(jnp.bfloat16),
                 preferred_element_type=jnp.float32) + bc1_ref[...]
    h2 = jnp.maximum(h2, 0.0)

    b = b_ref[...]
    a = jnp.bitwise_and(b, _GA - 1)
    s = jnp.right_shift(b, 7)
    sub = jax.lax.broadcasted_iota(jnp.int32, (_GA, t), 0)
    oa = (sub == a).astype(jnp.bfloat16)
    s0 = (s == 0).astype(jnp.float32)
    s1 = (s == 1).astype(jnp.float32)
    tmp = jnp.concatenate([
        h2 * s0,
        h2 * s1,
        s0,
        s1,
    ], axis=0).astype(jnp.bfloat16)
    out_ref[...] += jax.lax.dot_general(
        oa, tmp, (((1,), (1,)), ((), ())),
        preferred_element_type=jnp.float32)


def _head_kernel(p_ref, wf0t_ref, bf0_ref, wf1t_ref, bf1_ref, out_ref):
    p = p_ref[...]
    pooled = jnp.concatenate([p[:, 0:32], p[:, 32:64]], axis=0)
    counts = jnp.concatenate([p[:, 64:65], p[:, 65:66]], axis=0)
    pooled = pooled * (1.0 / jnp.maximum(counts, 1.0))
    h = jnp.dot(pooled, wf0t_ref[...],
                preferred_element_type=jnp.float32) + bf0_ref[...]
    h = jnp.maximum(h, 0.0)
    out_ref[...] = jnp.dot(h, wf1t_ref[...],
                           preferred_element_type=jnp.float32) + bf1_ref[...]


@functools.partial(jax.jit, static_argnames=("node_tile",))
def _forward(x, batch, wc0, bc0, wc1, bc1, wf0, bf0, wf1, bf1,
             node_tile=32768):
    n, cin = x.shape
    hdim = wc0.shape[0]
    hfc = wf1.shape[0]

    n_pad = ((n + node_tile - 1) // node_tile) * node_tile
    tiles = n_pad // node_tile

    x_t = x.T
    b_r = batch.astype(jnp.int32).reshape(1, n)
    if n_pad != n:
        x_t = jnp.pad(x_t, ((0, 0), (0, n_pad - n)))
        b_r = jnp.pad(b_r, ((0, 0), (0, n_pad - n)), constant_values=2 * _GA)

    wc0b = wc0.astype(jnp.bfloat16)
    bc0c = bc0.astype(jnp.float32).reshape(hdim, 1)
    wc1b = wc1.astype(jnp.bfloat16)
    bc1c = bc1.astype(jnp.float32).reshape(hdim, 1)

    flops = int(2 * n_pad * (cin * hdim + hdim * hdim + _GA * _NCOL))
    bytes_accessed = int(x_t.size * 4 + b_r.size * 4 + _GA * _NCOL * 4)
    cost = pl.CostEstimate(flops=flops, transcendentals=0,
                           bytes_accessed=bytes_accessed)

    partials = pl.pallas_call(
        _conv_pool_kernel,
        out_shape=jax.ShapeDtypeStruct((_GA, _NCOL), jnp.float32),
        grid=(tiles,),
        in_specs=[
            pl.BlockSpec((cin, node_tile), lambda i: (0, i)),
            pl.BlockSpec((1, node_tile), lambda i: (0, i)),
            pl.BlockSpec((hdim, cin), lambda i: (0, 0)),
            pl.BlockSpec((hdim, 1), lambda i: (0, 0)),
            pl.BlockSpec((hdim, hdim), lambda i: (0, 0)),
            pl.BlockSpec((hdim, 1), lambda i: (0, 0)),
        ],
        out_specs=pl.BlockSpec((_GA, _NCOL), lambda i: (0, 0)),
        compiler_params=pltpu.CompilerParams(
            dimension_semantics=("arbitrary",),
            vmem_limit_bytes=64 * 1024 * 1024,
        ),
        cost_estimate=cost,
    )(x_t, b_r, wc0b, bc0c, wc1b, bc1c)

    out = pl.pallas_call(
        _head_kernel,
        out_shape=jax.ShapeDtypeStruct((2 * _GA, hfc), jnp.float32),
    )(partials, wf0.T.astype(jnp.float32),
      bf0.astype(jnp.float32).reshape(1, hfc),
      wf1.T.astype(jnp.float32),
      bf1.astype(jnp.float32).reshape(1, hfc))
    return out


def kernel(x, edge_index, batch, wc0, bc0, wc1, bc1, wf0, bf0, wf1, bf1):
    del edge_index
    return _forward(x, batch, wc0, bc0, wc1, bc1, wf0, bf0, wf1, bf1)


# speedup vs baseline: 8.9961x; 1.0220x over previous
import functools

import jax
import jax.numpy as jnp
from jax.experimental import pallas as pl
from jax.experimental.pallas import tpu as pltpu

_GA = 128
_NCOL = 66


def _conv_pool_kernel(x_ref, b_ref, wc0_ref, bc0_ref, wc1_ref, bc1_ref,
                      out_ref):
    i = pl.program_id(0)

    @pl.when(i == 0)
    def _init():
        out_ref[...] = jnp.zeros_like(out_ref)

    t = x_ref.shape[1]
    h = jnp.dot(wc0_ref[...], x_ref[...].astype(jnp.bfloat16),
                preferred_element_type=jnp.float32) + bc0_ref[...]
    h = jnp.maximum(h, 0.0)
    h2 = jnp.dot(wc1_ref[...], h.astype(jnp.bfloat16),
                 preferred_element_type=jnp.float32) + bc1_ref[...]
    h2 = jnp.maximum(h2, 0.0)

    b = b_ref[...]
    a = jnp.bitwise_and(b, _GA - 1)
    s = jnp.right_shift(b, 7)
    sub = jax.lax.broadcasted_iota(jnp.int32, (_GA, t), 0)
    oa = (sub == a).astype(jnp.bfloat16)
    s0 = (s == 0).astype(jnp.float32)
    s1 = (s == 1).astype(jnp.float32)
    tmp = jnp.concatenate([
        h2 * s0,
        h2 * s1,
        s0,
        s1,
    ], axis=0).astype(jnp.bfloat16)
    out_ref[...] += jax.lax.dot_general(
        oa, tmp, (((1,), (1,)), ((), ())),
        preferred_element_type=jnp.float32)


def _head_kernel(p_ref, wf0t_ref, bf0_ref, wf1t_ref, bf1_ref, out_ref):
    p = p_ref[...]
    pooled = jnp.concatenate([p[:, 0:32], p[:, 32:64]], axis=0)
    counts = jnp.concatenate([p[:, 64:65], p[:, 65:66]], axis=0)
    pooled = pooled * (1.0 / jnp.maximum(counts, 1.0))
    h = jnp.dot(pooled, wf0t_ref[...],
                preferred_element_type=jnp.float32) + bf0_ref[...]
    h = jnp.maximum(h, 0.0)
    out_ref[...] = jnp.dot(h, wf1t_ref[...],
                           preferred_element_type=jnp.float32) + bf1_ref[...]


@functools.partial(jax.jit, static_argnames=("node_tile",))
def _forward(x, batch, wc0, bc0, wc1, bc1, wf0, bf0, wf1, bf1,
             node_tile=65536):
    n, cin = x.shape
    hdim = wc0.shape[0]
    hfc = wf1.shape[0]

    n_pad = ((n + node_tile - 1) // node_tile) * node_tile
    tiles = n_pad // node_tile

    x_t = x.T
    b_r = batch.astype(jnp.int32).reshape(1, n)
    if n_pad != n:
        x_t = jnp.pad(x_t, ((0, 0), (0, n_pad - n)))
        b_r = jnp.pad(b_r, ((0, 0), (0, n_pad - n)), constant_values=2 * _GA)

    wc0b = wc0.astype(jnp.bfloat16)
    bc0c = bc0.astype(jnp.float32).reshape(hdim, 1)
    wc1b = wc1.astype(jnp.bfloat16)
    bc1c = bc1.astype(jnp.float32).reshape(hdim, 1)

    flops = int(2 * n_pad * (cin * hdim + hdim * hdim + _GA * _NCOL))
    bytes_accessed = int(x_t.size * 4 + b_r.size * 4 + _GA * _NCOL * 4)
    cost = pl.CostEstimate(flops=flops, transcendentals=0,
                           bytes_accessed=bytes_accessed)

    partials = pl.pallas_call(
        _conv_pool_kernel,
        out_shape=jax.ShapeDtypeStruct((_GA, _NCOL), jnp.float32),
        grid=(tiles,),
        in_specs=[
            pl.BlockSpec((cin, node_tile), lambda i: (0, i)),
            pl.BlockSpec((1, node_tile), lambda i: (0, i)),
            pl.BlockSpec((hdim, cin), lambda i: (0, 0)),
            pl.BlockSpec((hdim, 1), lambda i: (0, 0)),
            pl.BlockSpec((hdim, hdim), lambda i: (0, 0)),
            pl.BlockSpec((hdim, 1), lambda i: (0, 0)),
        ],
        out_specs=pl.BlockSpec((_GA, _NCOL), lambda i: (0, 0)),
        compiler_params=pltpu.CompilerParams(
            dimension_semantics=("arbitrary",),
            vmem_limit_bytes=64 * 1024 * 1024,
        ),
        cost_estimate=cost,
    )(x_t, b_r, wc0b, bc0c, wc1b, bc1c)

    out = pl.pallas_call(
        _head_kernel,
        out_shape=jax.ShapeDtypeStruct((2 * _GA, hfc), jnp.float32),
    )(partials, wf0.T.astype(jnp.float32),
      bf0.astype(jnp.float32).reshape(1, hfc),
      wf1.T.astype(jnp.float32),
      bf1.astype(jnp.float32).reshape(1, hfc))
    return out


def kernel(x, edge_index, batch, wc0, bc0, wc1, bc1, wf0, bf0, wf1, bf1):
    del edge_index
    return _forward(x, batch, wc0, bc0, wc1, bc1, wf0, bf0, wf1, bf1)
